# baseline (speedup 1.0000x reference)
import functools

import jax
import jax.numpy as jnp
from jax import lax
from jax.experimental import pallas as pl
from jax.experimental.pallas import tpu as pltpu
from jax.experimental.pallas import tpu_sc as plsc

N_NODES = 10000
D = 128
E_EDGES = 320000
L_LEAF = 5000

NC = 2
NS = 16
NW = NC * NS

NP = 10240
EP = 327680
EPT = EP // NW
NBLK = EPT // 128
EBLK = 64
EPS = 344064
EPTS = EPS // NW
NBLK2 = EPTS // EBLK
NBUF = 2
LP = 8192
LPS = LP // NS
LPT = LP // NW
RPS = NP // NS

f32 = jnp.float32


@functools.lru_cache(maxsize=None)
def _sc_kernels():
    mesh = plsc.VectorSubcoreMesh(core_axis_name="c", subcore_axis_name="s",
                                  num_cores=NC, num_subcores=NS)

    @functools.partial(
        pl.kernel,
        out_type=jax.ShapeDtypeStruct((NC * NP, D), f32),
        mesh=mesh,
        scratch_types=[
            pltpu.VMEM((128, D), f32),
            pltpu.VMEM((NBLK, 128), jnp.int32),
            pltpu.VMEM_SHARED((NP, D), f32),
        ],
    )
    def _deg_kernel(dst2_hbm, zeros_hbm, ones_hbm, out_hbm,
                    ones_v, dst_v, acc_sh):
        cid = lax.axis_index("c")
        sid = lax.axis_index("s")
        wid = sid * NC + cid
        pltpu.sync_copy(zeros_hbm.at[pl.ds(sid * RPS, RPS)],
                        acc_sh.at[pl.ds(sid * RPS, RPS)])
        pltpu.sync_copy(ones_hbm, ones_v)
        pltpu.sync_copy(dst2_hbm.at[pl.ds(wid * NBLK, NBLK)], dst_v)
        plsc.subcore_barrier()

        @pl.loop(0, NBLK)
        def _(b):
            pltpu.sync_copy(ones_v, acc_sh.at[dst_v.at[b]], add=True)

        plsc.subcore_barrier()
        pltpu.sync_copy(acc_sh.at[pl.ds(sid * RPS, RPS)],
                        out_hbm.at[pl.ds(cid * NP + sid * RPS, RPS)])

    def _segsum_body(h_hbm, src64_hbm, dst64_hbm, zeros_hbm, out_hbm,
                     src_v, dst_v, bufs, sems):
        cid = lax.axis_index("c")
        sid = lax.axis_index("s")
        wid = sid * NC + cid

        def _gather(b, rows, sem):
            return pltpu.make_async_copy(
                h_hbm.at[src_v.at[pl.ds(b * EBLK, EBLK)]], rows, sem)

        pltpu.sync_copy(src64_hbm.at[pl.ds(wid * EPTS, EPTS)], src_v)
        pltpu.sync_copy(dst64_hbm.at[pl.ds(wid * NBLK2, NBLK2)], dst_v)
        return cid, sid, wid, _gather

    def _segsum_stream(_gather, dst_v, acc_sh, bufs, sems):
        for j in range(NBUF):
            _gather(j, bufs[j], sems[j]).start()

        @pl.loop(0, NBLK2 - NBUF, step=NBUF)
        def _(b):
            for j in range(NBUF):
                _gather(b + j, bufs[j], sems[j]).wait()
                pltpu.sync_copy(bufs[j], acc_sh.at[dst_v.at[b + j]], add=True)
                _gather(b + j + NBUF, bufs[j], sems[j]).start()

        for j in range(NBUF):
            _gather(NBLK2 - NBUF + j, bufs[j], sems[j]).wait()
            pltpu.sync_copy(bufs[j], acc_sh.at[dst_v.at[NBLK2 - NBUF + j]],
                            add=True)

    @functools.partial(
        pl.kernel,
        out_type=jax.ShapeDtypeStruct((NC * NP, D), f32),
        mesh=mesh,
        scratch_types=[
            pltpu.VMEM((EPTS,), jnp.int32),
            pltpu.VMEM((NBLK2, EBLK), jnp.int32),
            pltpu.VMEM((EBLK, D), f32),
            pltpu.VMEM((EBLK, D), f32),
            pltpu.VMEM_SHARED((NP, D), f32),
            pltpu.SemaphoreType.DMA,
            pltpu.SemaphoreType.DMA,
        ],
    )
    def _segsum_kernel(h_hbm, src_hbm, dst64_hbm, zeros_hbm, out_hbm,
                       src_v, dst_v, rows0, rows1,
                       acc_sh, sem0, sem1):
        bufs = (rows0, rows1)
        sems = (sem0, sem1)
        cid, sid, wid, _gather = _segsum_body(
            h_hbm, src_hbm, dst64_hbm, zeros_hbm, out_hbm,
            src_v, dst_v, bufs, sems)
        pltpu.sync_copy(zeros_hbm.at[pl.ds(sid * RPS, RPS)],
                        acc_sh.at[pl.ds(sid * RPS, RPS)])
        plsc.subcore_barrier()

        _segsum_stream(_gather, dst_v, acc_sh, bufs, sems)

        plsc.subcore_barrier()
        pltpu.sync_copy(acc_sh.at[pl.ds(sid * RPS, RPS)],
                        out_hbm.at[pl.ds(cid * NP + sid * RPS, RPS)])

    @functools.partial(
        pl.kernel,
        out_type=[jax.ShapeDtypeStruct((NC * NP, D), f32),
                  jax.ShapeDtypeStruct((NC * LP, D), f32),
                  jax.ShapeDtypeStruct((LP, D), f32)],
        mesh=mesh,
        scratch_types=[
            pltpu.VMEM((EPTS,), jnp.int32),
            pltpu.VMEM((NBLK2, EBLK), jnp.int32),
            pltpu.VMEM((EBLK, D), f32),
            pltpu.VMEM((EBLK, D), f32),
            pltpu.VMEM((EBLK,), jnp.int32),
            pltpu.VMEM_SHARED((NP, D), f32),
            pltpu.SemaphoreType.DMA,
            pltpu.SemaphoreType.DMA,
        ],
    )
    def _segsum_leaf_kernel(h_hbm, src_hbm, dst64_hbm, zeros_hbm, leaf2_hbm,
                            leaf_hbm, r_hbm, out_hbm, hl_hbm, rl_hbm,
                            src_v, dst_v, rows0, rows1,
                            lidx_v, acc_sh, sem0, sem1):
        bufs = (rows0, rows1)
        sems = (sem0, sem1)
        cid, sid, wid, _gather = _segsum_body(
            h_hbm, src_hbm, dst64_hbm, zeros_hbm, out_hbm,
            src_v, dst_v, bufs, sems)
        pltpu.sync_copy(zeros_hbm.at[pl.ds(sid * RPS, RPS)],
                        acc_sh.at[pl.ds(sid * RPS, RPS)])
        plsc.subcore_barrier()

        _segsum_stream(_gather, dst_v, acc_sh, bufs, sems)

        plsc.subcore_barrier()
        pltpu.sync_copy(acc_sh.at[pl.ds(sid * RPS, RPS)],
                        out_hbm.at[pl.ds(cid * NP + sid * RPS, RPS)])
        plsc.subcore_barrier()

        @pl.loop(0, LPS // EBLK)
        def _(k):
            base = sid * LPS + k * EBLK
            pltpu.sync_copy(leaf2_hbm.at[pl.ds(base, EBLK)], lidx_v)
            pltpu.async_copy(acc_sh.at[lidx_v], rows0, sem0).wait()
            pltpu.sync_copy(rows0, hl_hbm.at[pl.ds(cid * LP + base, EBLK)])

        @pl.loop(0, LPT // EBLK)
        def _(k):
            base = wid * LPT + k * EBLK
            pltpu.sync_copy(leaf_hbm.at[pl.ds(base, EBLK)], lidx_v)
            pltpu.async_copy(r_hbm.at[lidx_v], rows0, sem0).wait()
            pltpu.sync_copy(rows0, rl_hbm.at[pl.ds(base, EBLK)])

    return _deg_kernel, _segsum_kernel, _segsum_leaf_kernel


BLK = 1024
GRID = NP // BLK


def _t1_body(degA_ref, degB_ref, x_ref, r_ref, xt_ref):
    deg = jnp.max(degA_ref[...] + degB_ref[...], axis=1, keepdims=True)
    r = lax.rsqrt(jnp.maximum(deg, 1.0))
    rb = jnp.broadcast_to(r, (BLK, D))
    r_ref[...] = rb
    xt_ref[...] = x_ref[...] * rb


def _t1(degA, degB, x_p):
    return pl.pallas_call(
        _t1_body,
        grid=(GRID,),
        in_specs=[pl.BlockSpec((BLK, D), lambda i: (i, 0)),
                  pl.BlockSpec((BLK, D), lambda i: (i, 0)),
                  pl.BlockSpec((BLK, D), lambda i: (i, 0))],
        out_specs=[pl.BlockSpec((BLK, D), lambda i: (i, 0)),
                   pl.BlockSpec((BLK, D), lambda i: (i, 0))],
        out_shape=[jax.ShapeDtypeStruct((NP, D), f32),
                   jax.ShapeDtypeStruct((NP, D), f32)],
    )(degA, degB, x_p)


def _t2_body(aggA_ref, aggB_ref, r_ref, W_ref, b_ref, out_ref):
    r = r_ref[...]
    a = (aggA_ref[...] + aggB_ref[...]) * r
    o = jnp.dot(a, W_ref[...], preferred_element_type=f32) + b_ref[...]
    out_ref[...] = jnp.maximum(o, 0.0) * r


def _t2(aggA, aggB, r, W, b):
    return pl.pallas_call(
        _t2_body,
        grid=(GRID,),
        in_specs=[pl.BlockSpec((BLK, D), lambda i: (i, 0)),
                  pl.BlockSpec((BLK, D), lambda i: (i, 0)),
                  pl.BlockSpec((BLK, D), lambda i: (i, 0)),
                  pl.BlockSpec((D, D), lambda i: (0, 0)),
                  pl.BlockSpec((1, D), lambda i: (0, 0))],
        out_specs=pl.BlockSpec((BLK, D), lambda i: (i, 0)),
        out_shape=jax.ShapeDtypeStruct((NP, D), f32),
    )(aggA, aggB, r, W, b)


def _t3_body(aggA_ref, aggB_ref, r_ref, asum_ref):
    i = pl.program_id(0)
    a = (aggA_ref[...] + aggB_ref[...]) * r_ref[...]
    rows = i * BLK + lax.broadcasted_iota(jnp.int32, (BLK, D), 0)
    am = jnp.where(rows < N_NODES, a, 0.0)

    @pl.when(i == 0)
    def _():
        asum_ref[...] = jnp.zeros_like(asum_ref)

    asum_ref[...] += jnp.sum(am, axis=0, keepdims=True)


def _t3(aggA, aggB, r):
    return pl.pallas_call(
        _t3_body,
        grid=(GRID,),
        in_specs=[pl.BlockSpec((BLK, D), lambda i: (i, 0)),
                  pl.BlockSpec((BLK, D), lambda i: (i, 0)),
                  pl.BlockSpec((BLK, D), lambda i: (i, 0))],
        out_specs=pl.BlockSpec((1, D), lambda i: (0, 0)),
        out_shape=jax.ShapeDtypeStruct((1, D), f32),
    )(aggA, aggB, r)


def _t4_body(hlA_ref, hlB_ref, rl_ref, asum_ref, W2_ref, b2_ref, Wa_ref,
             Wb_ref, Wc_ref, we_ref, bs1_ref, Ws2_ref, bs2_ref, out_ref):
    graph = (jnp.dot(asum_ref[...], W2_ref[...], preferred_element_type=f32)
             * (1.0 / N_NODES) + b2_ref[...])
    const = (jnp.dot(we_ref[...], Wa_ref[...], preferred_element_type=f32)
             + jnp.dot(graph, Wc_ref[...], preferred_element_type=f32)
             + jnp.dot(b2_ref[...], Wb_ref[...], preferred_element_type=f32)
             + bs1_ref[...])
    W2b = jnp.dot(W2_ref[...], Wb_ref[...], preferred_element_type=f32)
    hpre = (hlA_ref[...] + hlB_ref[...]) * rl_ref[...]
    z = jnp.dot(hpre, W2b, preferred_element_type=f32)
    pre = jnp.maximum(z + const, 0.0)
    s = jnp.dot(pre, Ws2_ref[...], preferred_element_type=f32) + bs2_ref[...]
    rows = lax.broadcasted_iota(jnp.int32, (LP, 1), 0)
    mask = rows < L_LEAF
    logits = jnp.where(mask, s, jnp.full_like(s, -1e30))
    m = jnp.max(logits)
    p = jnp.where(mask, jnp.exp(logits - m), 0.0)
    out_ref[...] = p / jnp.sum(p)


def _t4(hlA, hlB, rl, asum, W2, b2, Wa, Wb, Wc, we, bs1, Ws2, bs2):
    full = lambda shp: pl.BlockSpec(shp, lambda i: (0, 0))
    return pl.pallas_call(
        _t4_body,
        grid=(1,),
        in_specs=[full((LP, D)), full((LP, D)), full((LP, D)), full((1, D)),
                  full((D, D)), full((1, D)), full((D, D)), full((D, D)),
                  full((D, D)), full((1, D)), full((1, D)), full((D, 1)),
                  full((1, 1))],
        out_specs=full((LP, 1)),
        out_shape=jax.ShapeDtypeStruct((LP, 1), f32),
    )(hlA, hlB, rl, asum, W2, b2, Wa, Wb, Wc, we, bs1, Ws2, bs2)


def kernel(x, edge_index, leaf_inds, word_emb, W1, b1, W2, b2, Ws1, bs1,
           Ws2, bs2):
    src = edge_index[0]
    dst = edge_index[1]
    pad_idx = jnp.full((EPS - E_EDGES,), N_NODES, jnp.int32)
    src_p = jnp.concatenate([src, pad_idx])
    dst_p = jnp.concatenate([dst, pad_idx])
    dst2 = dst_p[:EP].reshape(EP // 128, 128)
    dst64 = dst_p.reshape(EPS // EBLK, EBLK)
    leaf_p = jnp.concatenate(
        [leaf_inds, jnp.full((LP - L_LEAF,), N_NODES, jnp.int32)])
    x_p = jnp.concatenate([x, jnp.zeros((NP - N_NODES, D), f32)], axis=0)
    zerosND = jnp.zeros((NP, D), f32)
    onesND = jnp.ones((128, D), f32)

    deg_k, segsum_k, segsum_leaf_k = _sc_kernels()
    deg2 = deg_k(dst2, zerosND, onesND)
    r, xt = _t1(deg2[:NP], deg2[NP:], x_p)
    agg1 = segsum_k(xt, src_p, dst64, zerosND)
    h1t = _t2(agg1[:NP], agg1[NP:], r, W1, b1.reshape(1, D))
    agg2, hl, rl = segsum_leaf_k(h1t, src_p, dst64, zerosND, leaf_p, leaf_p, r)
    asum = _t3(agg2[:NP], agg2[NP:], r)
    out = _t4(hl[:LP], hl[LP:], rl, asum, W2, b2.reshape(1, D),
              Ws1[:D], Ws1[D:2 * D], Ws1[2 * D:], word_emb.reshape(1, D),
              bs1.reshape(1, D), Ws2, bs2.reshape(1, 1))
    return out[:L_LEAF]


# speedup vs baseline: 1.9557x; 1.9557x over previous
import functools

import jax
import jax.numpy as jnp
from jax import lax
from jax.experimental import pallas as pl
from jax.experimental.pallas import tpu as pltpu
from jax.experimental.pallas import tpu_sc as plsc

N_NODES = 10000
D = 128
E_EDGES = 320000
L_LEAF = 5000

NC = 2
NS = 16
NW = NC * NS

NP = 10240
EP = 327680
EPT = EP // NW
NBLK = EPT // 128
EBLK = 128
EPS = EP
EPTS = EPS // NW
NBLK2 = EPTS // EBLK
LP = 8192
LPS = LP // NS
LPT = LP // NW
RPS = NP // NS

f32 = jnp.float32


@functools.lru_cache(maxsize=None)
def _sc_kernels():
    mesh = plsc.VectorSubcoreMesh(core_axis_name="c", subcore_axis_name="s",
                                  num_cores=NC, num_subcores=NS)

    @functools.partial(
        pl.kernel,
        out_type=jax.ShapeDtypeStruct((NC * NP, D), f32),
        mesh=mesh,
        scratch_types=[
            pltpu.VMEM((128, D), f32),
            pltpu.VMEM((NBLK, 128), jnp.int32),
            pltpu.VMEM_SHARED((NP, D), f32),
        ],
    )
    def _deg_kernel(dst2_hbm, zeros_hbm, ones_hbm, out_hbm,
                    ones_v, dst_v, acc_sh):
        cid = lax.axis_index("c")
        sid = lax.axis_index("s")
        wid = sid * NC + cid
        pltpu.sync_copy(zeros_hbm.at[pl.ds(sid * RPS, RPS)],
                        acc_sh.at[pl.ds(sid * RPS, RPS)])
        pltpu.sync_copy(ones_hbm, ones_v)
        pltpu.sync_copy(dst2_hbm.at[pl.ds(wid * NBLK, NBLK)], dst_v)
        plsc.subcore_barrier()

        @pl.loop(0, NBLK)
        def _(b):
            pltpu.sync_copy(ones_v, acc_sh.at[dst_v.at[b]], add=True)

        plsc.subcore_barrier()
        pltpu.sync_copy(acc_sh.at[pl.ds(sid * RPS, RPS)],
                        out_hbm.at[pl.ds(cid * NP + sid * RPS, RPS)])

    def _segsum_body(h_hbm, src64_hbm, dst64_hbm, zeros_hbm, out_hbm,
                     src_v, dst_v, bufs, sems):
        cid = lax.axis_index("c")
        sid = lax.axis_index("s")
        wid = sid * NC + cid

        def _gather(b, rows, sem):
            return pltpu.make_async_copy(
                h_hbm.at[src_v.at[pl.ds(b * EBLK, EBLK)]], rows, sem)

        pltpu.sync_copy(src64_hbm.at[pl.ds(wid * EPTS, EPTS)], src_v)
        pltpu.sync_copy(dst64_hbm.at[pl.ds(wid * NBLK2, NBLK2)], dst_v)
        return cid, sid, wid, _gather

    def _segsum_stream(_gather, dst_v, acc_sh, bufs, sems):
        @pl.loop(0, NBLK2)
        def _(b):
            _gather(b, bufs[0], sems[0]).start()
            _gather(b, bufs[0], sems[0]).wait()
            pltpu.sync_copy(bufs[0], acc_sh.at[dst_v.at[b]], add=True)

    @functools.partial(
        pl.kernel,
        out_type=jax.ShapeDtypeStruct((NC * NP, D), f32),
        mesh=mesh,
        scratch_types=[
            pltpu.VMEM((EPTS,), jnp.int32),
            pltpu.VMEM((NBLK2, EBLK), jnp.int32),
            pltpu.VMEM((EBLK, D), f32),
            pltpu.VMEM_SHARED((NP, D), f32),
            pltpu.SemaphoreType.DMA,
        ],
    )
    def _segsum_kernel(h_hbm, src_hbm, dst64_hbm, zeros_hbm, out_hbm,
                       src_v, dst_v, rows0,
                       acc_sh, sem0):
        bufs = (rows0,)
        sems = (sem0,)
        cid, sid, wid, _gather = _segsum_body(
            h_hbm, src_hbm, dst64_hbm, zeros_hbm, out_hbm,
            src_v, dst_v, bufs, sems)
        pltpu.sync_copy(zeros_hbm.at[pl.ds(sid * RPS, RPS)],
                        acc_sh.at[pl.ds(sid * RPS, RPS)])
        plsc.subcore_barrier()

        _segsum_stream(_gather, dst_v, acc_sh, bufs, sems)

        plsc.subcore_barrier()
        pltpu.sync_copy(acc_sh.at[pl.ds(sid * RPS, RPS)],
                        out_hbm.at[pl.ds(cid * NP + sid * RPS, RPS)])

    @functools.partial(
        pl.kernel,
        out_type=[jax.ShapeDtypeStruct((NC * NP, D), f32),
                  jax.ShapeDtypeStruct((NC * LP, D), f32),
                  jax.ShapeDtypeStruct((LP, D), f32)],
        mesh=mesh,
        scratch_types=[
            pltpu.VMEM((EPTS,), jnp.int32),
            pltpu.VMEM((NBLK2, EBLK), jnp.int32),
            pltpu.VMEM((EBLK, D), f32),
            pltpu.VMEM((EBLK,), jnp.int32),
            pltpu.VMEM_SHARED((NP, D), f32),
            pltpu.SemaphoreType.DMA,
        ],
    )
    def _segsum_leaf_kernel(h_hbm, src_hbm, dst64_hbm, zeros_hbm, leaf2_hbm,
                            leaf_hbm, r_hbm, out_hbm, hl_hbm, rl_hbm,
                            src_v, dst_v, rows0,
                            lidx_v, acc_sh, sem0):
        bufs = (rows0,)
        sems = (sem0,)
        cid, sid, wid, _gather = _segsum_body(
            h_hbm, src_hbm, dst64_hbm, zeros_hbm, out_hbm,
            src_v, dst_v, bufs, sems)
        pltpu.sync_copy(zeros_hbm.at[pl.ds(sid * RPS, RPS)],
                        acc_sh.at[pl.ds(sid * RPS, RPS)])
        plsc.subcore_barrier()

        _segsum_stream(_gather, dst_v, acc_sh, bufs, sems)

        plsc.subcore_barrier()
        pltpu.sync_copy(acc_sh.at[pl.ds(sid * RPS, RPS)],
                        out_hbm.at[pl.ds(cid * NP + sid * RPS, RPS)])
        plsc.subcore_barrier()

        @pl.loop(0, LPS // EBLK)
        def _(k):
            base = sid * LPS + k * EBLK
            pltpu.sync_copy(leaf2_hbm.at[pl.ds(base, EBLK)], lidx_v)
            pltpu.async_copy(acc_sh.at[lidx_v], rows0, sem0).wait()
            pltpu.sync_copy(rows0, hl_hbm.at[pl.ds(cid * LP + base, EBLK)])

        @pl.loop(0, LPT // EBLK)
        def _(k):
            base = wid * LPT + k * EBLK
            pltpu.sync_copy(leaf_hbm.at[pl.ds(base, EBLK)], lidx_v)
            pltpu.async_copy(r_hbm.at[lidx_v], rows0, sem0).wait()
            pltpu.sync_copy(rows0, rl_hbm.at[pl.ds(base, EBLK)])

    return _deg_kernel, _segsum_kernel, _segsum_leaf_kernel


BLK = 1024
GRID = NP // BLK


def _t1_body(degA_ref, degB_ref, x_ref, r_ref, xt_ref):
    deg = jnp.max(degA_ref[...] + degB_ref[...], axis=1, keepdims=True)
    r = lax.rsqrt(jnp.maximum(deg, 1.0))
    rb = jnp.broadcast_to(r, (BLK, D))
    r_ref[...] = rb
    xt_ref[...] = x_ref[...] * rb


def _t1(degA, degB, x_p):
    return pl.pallas_call(
        _t1_body,
        grid=(GRID,),
        in_specs=[pl.BlockSpec((BLK, D), lambda i: (i, 0)),
                  pl.BlockSpec((BLK, D), lambda i: (i, 0)),
                  pl.BlockSpec((BLK, D), lambda i: (i, 0))],
        out_specs=[pl.BlockSpec((BLK, D), lambda i: (i, 0)),
                   pl.BlockSpec((BLK, D), lambda i: (i, 0))],
        out_shape=[jax.ShapeDtypeStruct((NP, D), f32),
                   jax.ShapeDtypeStruct((NP, D), f32)],
    )(degA, degB, x_p)


def _t2_body(aggA_ref, aggB_ref, r_ref, W_ref, b_ref, out_ref):
    r = r_ref[...]
    a = (aggA_ref[...] + aggB_ref[...]) * r
    o = jnp.dot(a, W_ref[...], preferred_element_type=f32) + b_ref[...]
    out_ref[...] = jnp.maximum(o, 0.0) * r


def _t2(aggA, aggB, r, W, b):
    return pl.pallas_call(
        _t2_body,
        grid=(GRID,),
        in_specs=[pl.BlockSpec((BLK, D), lambda i: (i, 0)),
                  pl.BlockSpec((BLK, D), lambda i: (i, 0)),
                  pl.BlockSpec((BLK, D), lambda i: (i, 0)),
                  pl.BlockSpec((D, D), lambda i: (0, 0)),
                  pl.BlockSpec((1, D), lambda i: (0, 0))],
        out_specs=pl.BlockSpec((BLK, D), lambda i: (i, 0)),
        out_shape=jax.ShapeDtypeStruct((NP, D), f32),
    )(aggA, aggB, r, W, b)


def _t3_body(aggA_ref, aggB_ref, r_ref, asum_ref):
    i = pl.program_id(0)
    a = (aggA_ref[...] + aggB_ref[...]) * r_ref[...]
    rows = i * BLK + lax.broadcasted_iota(jnp.int32, (BLK, D), 0)
    am = jnp.where(rows < N_NODES, a, 0.0)

    @pl.when(i == 0)
    def _():
        asum_ref[...] = jnp.zeros_like(asum_ref)

    asum_ref[...] += jnp.sum(am, axis=0, keepdims=True)


def _t3(aggA, aggB, r):
    return pl.pallas_call(
        _t3_body,
        grid=(GRID,),
        in_specs=[pl.BlockSpec((BLK, D), lambda i: (i, 0)),
                  pl.BlockSpec((BLK, D), lambda i: (i, 0)),
                  pl.BlockSpec((BLK, D), lambda i: (i, 0))],
        out_specs=pl.BlockSpec((1, D), lambda i: (0, 0)),
        out_shape=jax.ShapeDtypeStruct((1, D), f32),
    )(aggA, aggB, r)


def _t4_body(hlA_ref, hlB_ref, rl_ref, asum_ref, W2_ref, b2_ref, Wa_ref,
             Wb_ref, Wc_ref, we_ref, bs1_ref, Ws2_ref, bs2_ref, out_ref):
    graph = (jnp.dot(asum_ref[...], W2_ref[...], preferred_element_type=f32)
             * (1.0 / N_NODES) + b2_ref[...])
    const = (jnp.dot(we_ref[...], Wa_ref[...], preferred_element_type=f32)
             + jnp.dot(graph, Wc_ref[...], preferred_element_type=f32)
             + jnp.dot(b2_ref[...], Wb_ref[...], preferred_element_type=f32)
             + bs1_ref[...])
    W2b = jnp.dot(W2_ref[...], Wb_ref[...], preferred_element_type=f32)
    hpre = (hlA_ref[...] + hlB_ref[...]) * rl_ref[...]
    z = jnp.dot(hpre, W2b, preferred_element_type=f32)
    pre = jnp.maximum(z + const, 0.0)
    s = jnp.dot(pre, Ws2_ref[...], preferred_element_type=f32) + bs2_ref[...]
    rows = lax.broadcasted_iota(jnp.int32, (LP, 1), 0)
    mask = rows < L_LEAF
    logits = jnp.where(mask, s, jnp.full_like(s, -1e30))
    m = jnp.max(logits)
    p = jnp.where(mask, jnp.exp(logits - m), 0.0)
    out_ref[...] = p / jnp.sum(p)


def _t4(hlA, hlB, rl, asum, W2, b2, Wa, Wb, Wc, we, bs1, Ws2, bs2):
    full = lambda shp: pl.BlockSpec(shp, lambda i: (0, 0))
    return pl.pallas_call(
        _t4_body,
        grid=(1,),
        in_specs=[full((LP, D)), full((LP, D)), full((LP, D)), full((1, D)),
                  full((D, D)), full((1, D)), full((D, D)), full((D, D)),
                  full((D, D)), full((1, D)), full((1, D)), full((D, 1)),
                  full((1, 1))],
        out_specs=full((LP, 1)),
        out_shape=jax.ShapeDtypeStruct((LP, 1), f32),
    )(hlA, hlB, rl, asum, W2, b2, Wa, Wb, Wc, we, bs1, Ws2, bs2)


def kernel(x, edge_index, leaf_inds, word_emb, W1, b1, W2, b2, Ws1, bs1,
           Ws2, bs2):
    src = edge_index[0]
    dst = edge_index[1]
    pad_idx = jnp.full((EPS - E_EDGES,), N_NODES, jnp.int32)
    src_p = jnp.concatenate([src, pad_idx])
    dst_p = jnp.concatenate([dst, pad_idx])
    dst2 = dst_p[:EP].reshape(EP // 128, 128)
    dst64 = dst_p.reshape(EPS // EBLK, EBLK)
    leaf_p = jnp.concatenate(
        [leaf_inds, jnp.full((LP - L_LEAF,), N_NODES, jnp.int32)])
    x_p = jnp.concatenate([x, jnp.zeros((NP - N_NODES, D), f32)], axis=0)
    zerosND = jnp.zeros((NP, D), f32)
    onesND = jnp.ones((128, D), f32)

    deg_k, segsum_k, segsum_leaf_k = _sc_kernels()
    deg2 = deg_k(dst2, zerosND, onesND)
    r, xt = _t1(deg2[:NP], deg2[NP:], x_p)
    agg1 = segsum_k(xt, src_p, dst64, zerosND)
    h1t = _t2(agg1[:NP], agg1[NP:], r, W1, b1.reshape(1, D))
    agg2, hl, rl = segsum_leaf_k(h1t, src_p, dst64, zerosND, leaf_p, leaf_p, r)
    asum = _t3(agg2[:NP], agg2[NP:], r)
    out = _t4(hl[:LP], hl[LP:], rl, asum, W2, b2.reshape(1, D),
              Ws1[:D], Ws1[D:2 * D], Ws1[2 * D:], word_emb.reshape(1, D),
              bs1.reshape(1, D), Ws2, bs2.reshape(1, 1))
    return out[:L_LEAF]


# speedup vs baseline: 2.1859x; 1.1177x over previous
import functools

import jax
import jax.numpy as jnp
from jax import lax
from jax.experimental import pallas as pl
from jax.experimental.pallas import tpu as pltpu
from jax.experimental.pallas import tpu_sc as plsc

N_NODES = 10000
D = 128
E_EDGES = 320000
L_LEAF = 5000

NC = 2
NS = 16
NW = NC * NS

NP = 10240
EP = 327680
EPT = EP // NW
NBLK = EPT // 128
EBLK = 128
EPS = EP
EPTS = EPS // NW
NBLK2 = EPTS // EBLK
LP = 8192
LPS = LP // NS
LPT = LP // NW
RPS = NP // NS

f32 = jnp.float32


@functools.lru_cache(maxsize=None)
def _sc_kernels():
    mesh = plsc.VectorSubcoreMesh(core_axis_name="c", subcore_axis_name="s",
                                  num_cores=NC, num_subcores=NS)

    @functools.partial(
        pl.kernel,
        out_type=jax.ShapeDtypeStruct((NC * NP, D), f32),
        mesh=mesh,
        scratch_types=[
            pltpu.VMEM((128, D), f32),
            pltpu.VMEM((NBLK, 128), jnp.int32),
            pltpu.VMEM_SHARED((NP, D), f32),
        ],
    )
    def _deg_kernel(dst2_hbm, zeros_hbm, ones_hbm, out_hbm,
                    ones_v, dst_v, acc_sh):
        cid = lax.axis_index("c")
        sid = lax.axis_index("s")
        wid = sid * NC + cid
        pltpu.sync_copy(zeros_hbm.at[pl.ds(sid * RPS, RPS)],
                        acc_sh.at[pl.ds(sid * RPS, RPS)])
        pltpu.sync_copy(ones_hbm, ones_v)
        pltpu.sync_copy(dst2_hbm.at[pl.ds(wid * NBLK, NBLK)], dst_v)
        plsc.subcore_barrier()

        @pl.loop(0, NBLK)
        def _(b):
            pltpu.sync_copy(ones_v, acc_sh.at[dst_v.at[b]], add=True)

        plsc.subcore_barrier()
        pltpu.sync_copy(acc_sh.at[pl.ds(sid * RPS, RPS)],
                        out_hbm.at[pl.ds(cid * NP + sid * RPS, RPS)])

    def _segsum_body(h_hbm, src_hbm, dst64_hbm, zeros_hbm, out_hbm,
                     sidxs, dst_v, bufs, sems):
        cid = lax.axis_index("c")
        sid = lax.axis_index("s")
        wid = sid * NC + cid

        def _start(b, j):
            pltpu.sync_copy(src_hbm.at[pl.ds(wid * EPTS + b * EBLK, EBLK)],
                            sidxs[j])
            pltpu.make_async_copy(h_hbm.at[sidxs[j]], bufs[j], sems[j]).start()

        def _drain(b, j):
            pltpu.make_async_copy(h_hbm.at[sidxs[j]], bufs[j], sems[j]).wait()

        pltpu.sync_copy(dst64_hbm.at[pl.ds(wid * NBLK2, NBLK2)], dst_v)
        return cid, sid, wid, _start, _drain

    def _segsum_stream(_start, _drain, dst_v, acc_sh, bufs, sems):
        _start(0, 0)
        _start(1, 1)

        @pl.loop(0, NBLK2 - 2, step=2)
        def _(b):
            _drain(b, 0)
            pltpu.sync_copy(bufs[0], acc_sh.at[dst_v.at[b]], add=True)
            _start(b + 2, 0)
            _drain(b + 1, 1)
            pltpu.sync_copy(bufs[1], acc_sh.at[dst_v.at[b + 1]], add=True)
            _start(b + 3, 1)

        _drain(NBLK2 - 2, 0)
        pltpu.sync_copy(bufs[0], acc_sh.at[dst_v.at[NBLK2 - 2]], add=True)
        _drain(NBLK2 - 1, 1)
        pltpu.sync_copy(bufs[1], acc_sh.at[dst_v.at[NBLK2 - 1]], add=True)

    @functools.partial(
        pl.kernel,
        out_type=jax.ShapeDtypeStruct((NC * NP, D), f32),
        mesh=mesh,
        scratch_types=[
            pltpu.VMEM((EBLK,), jnp.int32),
            pltpu.VMEM((EBLK,), jnp.int32),
            pltpu.VMEM((NBLK2, EBLK), jnp.int32),
            pltpu.VMEM((EBLK, D), f32),
            pltpu.VMEM((EBLK, D), f32),
            pltpu.VMEM_SHARED((NP, D), f32),
            pltpu.SemaphoreType.DMA,
            pltpu.SemaphoreType.DMA,
        ],
    )
    def _segsum_kernel(h_hbm, src_hbm, dst64_hbm, zeros_hbm, out_hbm,
                       sidx0, sidx1, dst_v, rows0, rows1,
                       acc_sh, sem0, sem1):
        bufs = (rows0, rows1)
        sems = (sem0, sem1)
        cid, sid, wid, _start, _drain = _segsum_body(
            h_hbm, src_hbm, dst64_hbm, zeros_hbm, out_hbm,
            (sidx0, sidx1), dst_v, bufs, sems)
        pltpu.sync_copy(zeros_hbm.at[pl.ds(sid * RPS, RPS)],
                        acc_sh.at[pl.ds(sid * RPS, RPS)])
        plsc.subcore_barrier()

        _segsum_stream(_start, _drain, dst_v, acc_sh, bufs, sems)

        plsc.subcore_barrier()
        pltpu.sync_copy(acc_sh.at[pl.ds(sid * RPS, RPS)],
                        out_hbm.at[pl.ds(cid * NP + sid * RPS, RPS)])

    @functools.partial(
        pl.kernel,
        out_type=[jax.ShapeDtypeStruct((NC * NP, D), f32),
                  jax.ShapeDtypeStruct((NC * LP, D), f32),
                  jax.ShapeDtypeStruct((LP, D), f32)],
        mesh=mesh,
        scratch_types=[
            pltpu.VMEM((EBLK,), jnp.int32),
            pltpu.VMEM((EBLK,), jnp.int32),
            pltpu.VMEM((NBLK2, EBLK), jnp.int32),
            pltpu.VMEM((EBLK, D), f32),
            pltpu.VMEM((EBLK, D), f32),
            pltpu.VMEM((EBLK,), jnp.int32),
            pltpu.VMEM_SHARED((NP, D), f32),
            pltpu.SemaphoreType.DMA,
            pltpu.SemaphoreType.DMA,
        ],
    )
    def _segsum_leaf_kernel(h_hbm, src_hbm, dst64_hbm, zeros_hbm, leaf2_hbm,
                            leaf_hbm, r_hbm, out_hbm, hl_hbm, rl_hbm,
                            sidx0, sidx1, dst_v, rows0, rows1,
                            lidx_v, acc_sh, sem0, sem1):
        bufs = (rows0, rows1)
        sems = (sem0, sem1)
        cid, sid, wid, _start, _drain = _segsum_body(
            h_hbm, src_hbm, dst64_hbm, zeros_hbm, out_hbm,
            (sidx0, sidx1), dst_v, bufs, sems)
        pltpu.sync_copy(zeros_hbm.at[pl.ds(sid * RPS, RPS)],
                        acc_sh.at[pl.ds(sid * RPS, RPS)])
        plsc.subcore_barrier()

        _segsum_stream(_start, _drain, dst_v, acc_sh, bufs, sems)

        plsc.subcore_barrier()
        pltpu.sync_copy(acc_sh.at[pl.ds(sid * RPS, RPS)],
                        out_hbm.at[pl.ds(cid * NP + sid * RPS, RPS)])
        plsc.subcore_barrier()

        @pl.loop(0, LPS // EBLK)
        def _(k):
            base = sid * LPS + k * EBLK
            pltpu.sync_copy(leaf2_hbm.at[pl.ds(base, EBLK)], lidx_v)
            pltpu.async_copy(acc_sh.at[lidx_v], rows0, sem0).wait()
            pltpu.sync_copy(rows0, hl_hbm.at[pl.ds(cid * LP + base, EBLK)])

        @pl.loop(0, LPT // EBLK)
        def _(k):
            base = wid * LPT + k * EBLK
            pltpu.sync_copy(leaf_hbm.at[pl.ds(base, EBLK)], lidx_v)
            pltpu.async_copy(r_hbm.at[lidx_v], rows0, sem0).wait()
            pltpu.sync_copy(rows0, rl_hbm.at[pl.ds(base, EBLK)])

    return _deg_kernel, _segsum_kernel, _segsum_leaf_kernel


BLK = 1024
GRID = NP // BLK


def _t1_body(degA_ref, degB_ref, x_ref, r_ref, xt_ref):
    deg = jnp.max(degA_ref[...] + degB_ref[...], axis=1, keepdims=True)
    r = lax.rsqrt(jnp.maximum(deg, 1.0))
    rb = jnp.broadcast_to(r, (BLK, D))
    r_ref[...] = rb
    xt_ref[...] = x_ref[...] * rb


def _t1(degA, degB, x_p):
    return pl.pallas_call(
        _t1_body,
        grid=(GRID,),
        in_specs=[pl.BlockSpec((BLK, D), lambda i: (i, 0)),
                  pl.BlockSpec((BLK, D), lambda i: (i, 0)),
                  pl.BlockSpec((BLK, D), lambda i: (i, 0))],
        out_specs=[pl.BlockSpec((BLK, D), lambda i: (i, 0)),
                   pl.BlockSpec((BLK, D), lambda i: (i, 0))],
        out_shape=[jax.ShapeDtypeStruct((NP, D), f32),
                   jax.ShapeDtypeStruct((NP, D), f32)],
    )(degA, degB, x_p)


def _t2_body(aggA_ref, aggB_ref, r_ref, W_ref, b_ref, out_ref):
    r = r_ref[...]
    a = (aggA_ref[...] + aggB_ref[...]) * r
    o = jnp.dot(a, W_ref[...], preferred_element_type=f32) + b_ref[...]
    out_ref[...] = jnp.maximum(o, 0.0) * r


def _t2(aggA, aggB, r, W, b):
    return pl.pallas_call(
        _t2_body,
        grid=(GRID,),
        in_specs=[pl.BlockSpec((BLK, D), lambda i: (i, 0)),
                  pl.BlockSpec((BLK, D), lambda i: (i, 0)),
                  pl.BlockSpec((BLK, D), lambda i: (i, 0)),
                  pl.BlockSpec((D, D), lambda i: (0, 0)),
                  pl.BlockSpec((1, D), lambda i: (0, 0))],
        out_specs=pl.BlockSpec((BLK, D), lambda i: (i, 0)),
        out_shape=jax.ShapeDtypeStruct((NP, D), f32),
    )(aggA, aggB, r, W, b)


def _t3_body(aggA_ref, aggB_ref, r_ref, asum_ref):
    i = pl.program_id(0)
    a = (aggA_ref[...] + aggB_ref[...]) * r_ref[...]
    rows = i * BLK + lax.broadcasted_iota(jnp.int32, (BLK, D), 0)
    am = jnp.where(rows < N_NODES, a, 0.0)

    @pl.when(i == 0)
    def _():
        asum_ref[...] = jnp.zeros_like(asum_ref)

    asum_ref[...] += jnp.sum(am, axis=0, keepdims=True)


def _t3(aggA, aggB, r):
    return pl.pallas_call(
        _t3_body,
        grid=(GRID,),
        in_specs=[pl.BlockSpec((BLK, D), lambda i: (i, 0)),
                  pl.BlockSpec((BLK, D), lambda i: (i, 0)),
                  pl.BlockSpec((BLK, D), lambda i: (i, 0))],
        out_specs=pl.BlockSpec((1, D), lambda i: (0, 0)),
        out_shape=jax.ShapeDtypeStruct((1, D), f32),
    )(aggA, aggB, r)


def _t4_body(hlA_ref, hlB_ref, rl_ref, asum_ref, W2_ref, b2_ref, Wa_ref,
             Wb_ref, Wc_ref, we_ref, bs1_ref, Ws2_ref, bs2_ref, out_ref):
    graph = (jnp.dot(asum_ref[...], W2_ref[...], preferred_element_type=f32)
             * (1.0 / N_NODES) + b2_ref[...])
    const = (jnp.dot(we_ref[...], Wa_ref[...], preferred_element_type=f32)
             + jnp.dot(graph, Wc_ref[...], preferred_element_type=f32)
             + jnp.dot(b2_ref[...], Wb_ref[...], preferred_element_type=f32)
             + bs1_ref[...])
    W2b = jnp.dot(W2_ref[...], Wb_ref[...], preferred_element_type=f32)
    hpre = (hlA_ref[...] + hlB_ref[...]) * rl_ref[...]
    z = jnp.dot(hpre, W2b, preferred_element_type=f32)
    pre = jnp.maximum(z + const, 0.0)
    s = jnp.dot(pre, Ws2_ref[...], preferred_element_type=f32) + bs2_ref[...]
    rows = lax.broadcasted_iota(jnp.int32, (LP, 1), 0)
    mask = rows < L_LEAF
    logits = jnp.where(mask, s, jnp.full_like(s, -1e30))
    m = jnp.max(logits)
    p = jnp.where(mask, jnp.exp(logits - m), 0.0)
    out_ref[...] = p / jnp.sum(p)


def _t4(hlA, hlB, rl, asum, W2, b2, Wa, Wb, Wc, we, bs1, Ws2, bs2):
    full = lambda shp: pl.BlockSpec(shp, lambda i: (0, 0))
    return pl.pallas_call(
        _t4_body,
        grid=(1,),
        in_specs=[full((LP, D)), full((LP, D)), full((LP, D)), full((1, D)),
                  full((D, D)), full((1, D)), full((D, D)), full((D, D)),
                  full((D, D)), full((1, D)), full((1, D)), full((D, 1)),
                  full((1, 1))],
        out_specs=full((LP, 1)),
        out_shape=jax.ShapeDtypeStruct((LP, 1), f32),
    )(hlA, hlB, rl, asum, W2, b2, Wa, Wb, Wc, we, bs1, Ws2, bs2)


def kernel(x, edge_index, leaf_inds, word_emb, W1, b1, W2, b2, Ws1, bs1,
           Ws2, bs2):
    src = edge_index[0]
    dst = edge_index[1]
    pad_idx = jnp.full((EPS - E_EDGES,), N_NODES, jnp.int32)
    src_p = jnp.concatenate([src, pad_idx])
    dst_p = jnp.concatenate([dst, pad_idx])
    dst2 = dst_p[:EP].reshape(EP // 128, 128)
    dst64 = dst_p.reshape(EPS // EBLK, EBLK)
    leaf_p = jnp.concatenate(
        [leaf_inds, jnp.full((LP - L_LEAF,), N_NODES, jnp.int32)])
    x_p = jnp.concatenate([x, jnp.zeros((NP - N_NODES, D), f32)], axis=0)
    zerosND = jnp.zeros((NP, D), f32)
    onesND = jnp.ones((128, D), f32)

    deg_k, segsum_k, segsum_leaf_k = _sc_kernels()
    deg2 = deg_k(dst2, zerosND, onesND)
    r, xt = _t1(deg2[:NP], deg2[NP:], x_p)
    agg1 = segsum_k(xt, src_p, dst64, zerosND)
    h1t = _t2(agg1[:NP], agg1[NP:], r, W1, b1.reshape(1, D))
    agg2, hl, rl = segsum_leaf_k(h1t, src_p, dst64, zerosND, leaf_p, leaf_p, r)
    asum = _t3(agg2[:NP], agg2[NP:], r)
    out = _t4(hl[:LP], hl[LP:], rl, asum, W2, b2.reshape(1, D),
              Ws1[:D], Ws1[D:2 * D], Ws1[2 * D:], word_emb.reshape(1, D),
              bs1.reshape(1, D), Ws2, bs2.reshape(1, 1))
    return out[:L_LEAF]
